# Initial kernel scaffold; baseline (speedup 1.0000x reference)
import jax
import jax.numpy as jnp
from jax.experimental import pallas as pl


def kernel(index):
    raise NotImplementedError("write your pallas kernel here")


# speedup vs baseline: 8.2327x; 8.2327x over previous
import jax
import jax.numpy as jnp
from jax.experimental import pallas as pl
from jax.experimental.pallas import tpu as pltpu

_B = 8192
_BR = 256


def _body(rows_ref, cols_ref, out_ref):
    xr = rows_ref[:, :]
    t1r = jnp.floor(xr)
    t2r = jnp.ceil(xr)
    i1r = jnp.clip(t1r.astype(jnp.int32), 0, _B - 1)
    i2r = jnp.clip(t2r.astype(jnp.int32), 0, _B - 1)

    xc = cols_ref[:, :]
    t1c = jnp.floor(xc)
    tc = jnp.where(jnp.ceil(xc) != t1c, xc - t1c, 0.0)

    col = jax.lax.broadcasted_iota(jnp.int32, (_BR, _B), 1)
    out_ref[:, :] = (jnp.where(col == i1r, tc, 0.0)
                     + jnp.where(col == i2r, 1.0 - tc, 0.0))


def kernel(index):
    idx_rows = index.reshape(_B, 1)
    idx_cols = index.reshape(1, _B)
    return pl.pallas_call(
        _body,
        grid=(_B // _BR,),
        in_specs=[
            pl.BlockSpec((_BR, 1), lambda i: (i, 0)),
            pl.BlockSpec((1, _B), lambda i: (0, 0)),
        ],
        out_specs=pl.BlockSpec((_BR, _B), lambda i: (i, 0)),
        out_shape=jax.ShapeDtypeStruct((_B, _B), index.dtype),
        compiler_params=pltpu.CompilerParams(
            dimension_semantics=("parallel",),
        ),
    )(idx_rows, idx_cols)


# speedup vs baseline: 8.4781x; 1.0298x over previous
import jax
import jax.numpy as jnp
from jax.experimental import pallas as pl
from jax.experimental.pallas import tpu as pltpu

_B = 8192
_BR = 256


def _body(rows_ref, cols_ref, out_ref):
    xr = rows_ref[:, :]
    t1r = jnp.floor(xr)
    t2r = jnp.ceil(xr)
    i1r = t1r.astype(jnp.int32)
    i2r = jnp.minimum(t2r.astype(jnp.int32), _B - 1)

    xc = cols_ref[:, :]
    t1c = jnp.floor(xc)
    tc = jnp.where(jnp.ceil(xc) != t1c, xc - t1c, 0.0)
    w2 = 1.0 - tc

    col = jax.lax.broadcasted_iota(jnp.int32, (8, _B), 1)
    for g in range(_BR // 8):
        s = slice(g * 8, (g + 1) * 8)
        a = col == i1r[s, :]
        b = col == i2r[s, :]
        out_ref[s, :] = jnp.where(a, tc, 0.0) + jnp.where(b, w2, 0.0)


def kernel(index):
    idx_rows = index.reshape(_B, 1)
    idx_cols = index.reshape(1, _B)
    return pl.pallas_call(
        _body,
        grid=(_B // _BR,),
        in_specs=[
            pl.BlockSpec((_BR, 1), lambda i: (i, 0)),
            pl.BlockSpec((1, _B), lambda i: (0, 0)),
        ],
        out_specs=pl.BlockSpec((_BR, _B), lambda i: (i, 0)),
        out_shape=jax.ShapeDtypeStruct((_B, _B), index.dtype),
        compiler_params=pltpu.CompilerParams(
            dimension_semantics=("parallel",),
        ),
    )(idx_rows, idx_cols)
